# baseline (speedup 1.0000x reference)
import functools

import jax
import jax.numpy as jnp
from jax import lax
from jax.experimental import pallas as pl
from jax.experimental.pallas import tpu as pltpu
from jax.experimental.pallas import tpu_sc as plsc

N = 10000
E = 320000
D_IN = 128
HID = 128
CLS = 16

NC = 2
NS = 16
NW = NC * NS
EPW = E // NW
K = 64
NCHUNK = 160
EPW_PAD = NCHUNK * K
NPAD = 16

RPT = 640
RPT_LAST = N - (NS - 1) * RPT


def _rows_copy(src_of, dst_of, sid):
  start = pl.multiple_of(sid * RPT, 8)

  @pl.when(sid < NS - 1)
  def _():
    pltpu.sync_copy(src_of(start, RPT), dst_of(start, RPT))

  @pl.when(sid == NS - 1)
  def _():
    pltpu.sync_copy(src_of(start, RPT_LAST), dst_of(start, RPT_LAST))

_MESH = plsc.VectorSubcoreMesh(
    core_axis_name="c", subcore_axis_name="s", num_cores=NC, num_subcores=NS
)
_SC_PARAMS = pltpu.CompilerParams(use_tc_tiling_on_sc=False)


def _make_sc_agg(F):
  @functools.partial(
      pl.kernel,
      out_type=jax.ShapeDtypeStruct((NC, N, F), jnp.float32),
      mesh=_MESH,
      compiler_params=_SC_PARAMS,
      scratch_types=[
          pltpu.VMEM((NCHUNK, K), jnp.int32),
          pltpu.VMEM((NCHUNK, K), jnp.int32),
          pltpu.VMEM((K, F), jnp.float32),
          pltpu.VMEM_SHARED((N + NPAD, F), jnp.float32),
          pltpu.SemaphoreType.DMA,
      ],
  )
  def agg(h_hbm, src_hbm, dst_hbm, zero_hbm, out_hbm,
          src_v, dst_v, rows_v, acc_sh, sem):
    cid = lax.axis_index("c")
    sid = lax.axis_index("s")
    wid = cid * NS + sid
    pltpu.sync_copy(src_hbm.at[wid], src_v)
    pltpu.sync_copy(dst_hbm.at[wid], dst_v)
    _rows_copy(lambda s, n: zero_hbm.at[pl.ds(s, n)],
               lambda s, n: acc_sh.at[pl.ds(s, n)], sid)
    plsc.subcore_barrier()

    def body(j, carry):
      pltpu.async_copy(h_hbm.at[src_v.at[j]], rows_v, sem).wait()
      pltpu.sync_copy(rows_v, acc_sh.at[dst_v.at[j]], add=True)
      return carry

    lax.fori_loop(0, NCHUNK, body, 0)
    plsc.subcore_barrier()
    _rows_copy(lambda s, n: acc_sh.at[pl.ds(s, n)],
               lambda s, n: out_hbm.at[cid].at[pl.ds(s, n)], sid)

  return agg


_sc_agg128 = _make_sc_agg(HID)
_sc_agg16 = _make_sc_agg(CLS)


@functools.partial(
    pl.kernel,
    out_type=jax.ShapeDtypeStruct((NC, N, 16), jnp.float32),
    mesh=_MESH,
    compiler_params=_SC_PARAMS,
    scratch_types=[
        pltpu.VMEM((NCHUNK, K), jnp.int32),
        pltpu.VMEM((K, 16), jnp.float32),
        pltpu.VMEM_SHARED((N + NPAD, 16), jnp.float32),
    ],
)
def _sc_deg(dst_hbm, zero_hbm, out_hbm, dst_v, ones_v, acc_sh):
  cid = lax.axis_index("c")
  sid = lax.axis_index("s")
  wid = cid * NS + sid
  pltpu.sync_copy(dst_hbm.at[wid], dst_v)
  _rows_copy(lambda s, n: zero_hbm.at[pl.ds(s, n)],
             lambda s, n: acc_sh.at[pl.ds(s, n)], sid)
  for j in range(K):
    ones_v[j, :] = jnp.full((16,), 1.0, jnp.float32)
  plsc.subcore_barrier()

  def body(j, carry):
    pltpu.sync_copy(ones_v, acc_sh.at[dst_v.at[j]], add=True)
    return carry

  lax.fori_loop(0, NCHUNK, body, 0)
  plsc.subcore_barrier()
  _rows_copy(lambda s, n: acc_sh.at[pl.ds(s, n)],
             lambda s, n: out_hbm.at[cid].at[pl.ds(s, n)], sid)


_B = 1000


def _tc1_body(x_ref, w_ref, deg_ref, z_ref, dinv_ref):
  deg = deg_ref[0] + deg_ref[1] + 1.0
  dinv = lax.rsqrt(deg)
  dinv_ref[...] = dinv
  z = jnp.dot(x_ref[...], w_ref[...], preferred_element_type=jnp.float32)
  z_ref[...] = z * dinv[:, :1]


def _tc1(x, W1, degs):
  return pl.pallas_call(
      _tc1_body,
      grid=(N // _B,),
      in_specs=[
          pl.BlockSpec((_B, D_IN), lambda i: (i, 0)),
          pl.BlockSpec((D_IN, HID), lambda i: (0, 0)),
          pl.BlockSpec((NC, _B, 16), lambda i: (0, i, 0)),
      ],
      out_specs=[
          pl.BlockSpec((_B, HID), lambda i: (i, 0)),
          pl.BlockSpec((_B, 16), lambda i: (i, 0)),
      ],
      out_shape=[
          jax.ShapeDtypeStruct((N, HID), jnp.float32),
          jax.ShapeDtypeStruct((N, 16), jnp.float32),
      ],
  )(x, W1, degs)


def _tc_mid_body(acc_ref, zp_ref, dinv_ref, w_ref, b_ref, out_ref):
  dinv = dinv_ref[...][:, :1]
  s = acc_ref[0] + acc_ref[1] + zp_ref[...]
  y = jnp.maximum(s * dinv + b_ref[...], 0.0)
  z = jnp.dot(y, w_ref[...], preferred_element_type=jnp.float32)
  out_ref[...] = z * dinv


def _tc_mid(acc, zp, dinv, W, b, f_out):
  return pl.pallas_call(
      _tc_mid_body,
      grid=(N // _B,),
      in_specs=[
          pl.BlockSpec((NC, _B, HID), lambda i: (0, i, 0)),
          pl.BlockSpec((_B, HID), lambda i: (i, 0)),
          pl.BlockSpec((_B, 16), lambda i: (i, 0)),
          pl.BlockSpec((HID, f_out), lambda i: (0, 0)),
          pl.BlockSpec((1, HID), lambda i: (0, 0)),
      ],
      out_specs=pl.BlockSpec((_B, f_out), lambda i: (i, 0)),
      out_shape=jax.ShapeDtypeStruct((N, f_out), jnp.float32),
  )(acc, zp, dinv, W, b)


def _tc_fin_body(acc_ref, zp_ref, dinv_ref, b_ref, out_ref):
  dinv = dinv_ref[...][:, :1]
  s = acc_ref[0] + acc_ref[1] + zp_ref[...]
  out_ref[...] = jnp.maximum(s * dinv + b_ref[...], 0.0)


def _tc_fin(acc, zp, dinv, b):
  return pl.pallas_call(
      _tc_fin_body,
      grid=(N // _B,),
      in_specs=[
          pl.BlockSpec((NC, _B, CLS), lambda i: (0, i, 0)),
          pl.BlockSpec((_B, CLS), lambda i: (i, 0)),
          pl.BlockSpec((_B, 16), lambda i: (i, 0)),
          pl.BlockSpec((1, CLS), lambda i: (0, 0)),
      ],
      out_specs=pl.BlockSpec((_B, CLS), lambda i: (i, 0)),
      out_shape=jax.ShapeDtypeStruct((N, CLS), jnp.float32),
  )(acc, zp, dinv, b)


def kernel(x, edge_index, W1, b1, W2, b2, W3, b3):
  pad_n = EPW_PAD - EPW
  src_pad = jnp.zeros((NW, pad_n), jnp.int32)
  dst_pad = jnp.broadcast_to(
      (N + jnp.arange(NW, dtype=jnp.int32) % NS)[:, None], (NW, pad_n))
  src = jnp.concatenate(
      [edge_index[0].reshape(NW, EPW), src_pad], 1).reshape(NW, NCHUNK, K)
  dst = jnp.concatenate(
      [edge_index[1].reshape(NW, EPW), dst_pad], 1).reshape(NW, NCHUNK, K)
  zeros128 = jnp.zeros((N, HID), jnp.float32)
  zeros16 = jnp.zeros((N, 16), jnp.float32)

  degs = _sc_deg(dst, zeros16)
  z1p, dinv = _tc1(x, W1, degs)
  acc1 = _sc_agg128(z1p, src, dst, zeros128)
  z2p = _tc_mid(acc1, z1p, dinv, W2, b1.reshape(1, HID), HID)
  acc2 = _sc_agg128(z2p, src, dst, zeros128)
  z3p = _tc_mid(acc2, z2p, dinv, W3, b2.reshape(1, HID), CLS)
  acc3 = _sc_agg16(z3p, src, dst, zeros16)
  return _tc_fin(acc3, z3p, dinv, b3.reshape(1, CLS))


# speedup vs baseline: 1.1295x; 1.1295x over previous
import functools

import jax
import jax.numpy as jnp
from jax import lax
from jax.experimental import pallas as pl
from jax.experimental.pallas import tpu as pltpu
from jax.experimental.pallas import tpu_sc as plsc

N = 10000
E = 320000
D_IN = 128
HID = 128
CLS = 16

NC = 2
NS = 16
NW = NC * NS
EPW = E // NW
K = 128
NCHUNK = 80
EPW_PAD = NCHUNK * K
TRASH_PER_TILE = 80
NPAD = NS * TRASH_PER_TILE

RPT = 640
RPT_LAST = N - (NS - 1) * RPT


def _rows_copy(src_of, dst_of, sid):
  start = pl.multiple_of(sid * RPT, 8)

  @pl.when(sid < NS - 1)
  def _():
    pltpu.sync_copy(src_of(start, RPT), dst_of(start, RPT))

  @pl.when(sid == NS - 1)
  def _():
    pltpu.sync_copy(src_of(start, RPT_LAST), dst_of(start, RPT_LAST))

_MESH = plsc.VectorSubcoreMesh(
    core_axis_name="c", subcore_axis_name="s", num_cores=NC, num_subcores=NS
)
_SC_PARAMS = pltpu.CompilerParams(use_tc_tiling_on_sc=False)


def _make_sc_agg(F):
  @functools.partial(
      pl.kernel,
      out_type=jax.ShapeDtypeStruct((NC, N, F), jnp.float32),
      mesh=_MESH,
      compiler_params=_SC_PARAMS,
      scratch_types=[
          pltpu.VMEM((NCHUNK, K), jnp.int32),
          pltpu.VMEM((NCHUNK, K), jnp.int32),
          pltpu.VMEM((K, F), jnp.float32),
          pltpu.VMEM_SHARED((N + NPAD, F), jnp.float32),
          pltpu.SemaphoreType.DMA,
      ],
  )
  def agg(h_hbm, src_hbm, dst_hbm, zero_hbm, out_hbm,
          src_v, dst_v, rows_v, acc_sh, sem):
    cid = lax.axis_index("c")
    sid = lax.axis_index("s")
    wid = cid * NS + sid
    pltpu.sync_copy(src_hbm.at[wid], src_v)
    pltpu.sync_copy(dst_hbm.at[wid], dst_v)
    _rows_copy(lambda s, n: zero_hbm.at[pl.ds(s, n)],
               lambda s, n: acc_sh.at[pl.ds(s, n)], sid)
    plsc.subcore_barrier()

    def body(j, carry):
      pltpu.async_copy(h_hbm.at[src_v.at[j]], rows_v, sem).wait()
      pltpu.sync_copy(rows_v, acc_sh.at[dst_v.at[j]], add=True)
      return carry

    lax.fori_loop(0, NCHUNK, body, 0)
    plsc.subcore_barrier()
    _rows_copy(lambda s, n: acc_sh.at[pl.ds(s, n)],
               lambda s, n: out_hbm.at[cid].at[pl.ds(s, n)], sid)

  return agg


_sc_agg128 = _make_sc_agg(HID)
_sc_agg16 = _make_sc_agg(CLS)


@functools.partial(
    pl.kernel,
    out_type=jax.ShapeDtypeStruct((NC, N, 16), jnp.float32),
    mesh=_MESH,
    compiler_params=_SC_PARAMS,
    scratch_types=[
        pltpu.VMEM((NCHUNK, K), jnp.int32),
        pltpu.VMEM((K, 16), jnp.float32),
        pltpu.VMEM_SHARED((N + NPAD, 16), jnp.float32),
    ],
)
def _sc_deg(dst_hbm, zero_hbm, out_hbm, dst_v, ones_v, acc_sh):
  cid = lax.axis_index("c")
  sid = lax.axis_index("s")
  wid = cid * NS + sid
  pltpu.sync_copy(dst_hbm.at[wid], dst_v)
  _rows_copy(lambda s, n: zero_hbm.at[pl.ds(s, n)],
             lambda s, n: acc_sh.at[pl.ds(s, n)], sid)
  for j in range(K):
    ones_v[j, :] = jnp.full((16,), 1.0, jnp.float32)
  plsc.subcore_barrier()

  def body(j, carry):
    pltpu.sync_copy(ones_v, acc_sh.at[dst_v.at[j]], add=True)
    return carry

  lax.fori_loop(0, NCHUNK, body, 0)
  plsc.subcore_barrier()
  _rows_copy(lambda s, n: acc_sh.at[pl.ds(s, n)],
             lambda s, n: out_hbm.at[cid].at[pl.ds(s, n)], sid)


_B = 1000


def _tc1_body(x_ref, w_ref, deg_ref, z_ref, dinv_ref):
  deg = deg_ref[0] + deg_ref[1] + 1.0
  dinv = lax.rsqrt(deg)
  dinv_ref[...] = dinv
  z = jnp.dot(x_ref[...], w_ref[...], preferred_element_type=jnp.float32)
  z_ref[...] = z * dinv[:, :1]


def _tc1(x, W1, degs):
  return pl.pallas_call(
      _tc1_body,
      grid=(N // _B,),
      in_specs=[
          pl.BlockSpec((_B, D_IN), lambda i: (i, 0)),
          pl.BlockSpec((D_IN, HID), lambda i: (0, 0)),
          pl.BlockSpec((NC, _B, 16), lambda i: (0, i, 0)),
      ],
      out_specs=[
          pl.BlockSpec((_B, HID), lambda i: (i, 0)),
          pl.BlockSpec((_B, 16), lambda i: (i, 0)),
      ],
      out_shape=[
          jax.ShapeDtypeStruct((N, HID), jnp.float32),
          jax.ShapeDtypeStruct((N, 16), jnp.float32),
      ],
  )(x, W1, degs)


def _tc_mid_body(acc_ref, zp_ref, dinv_ref, w_ref, b_ref, out_ref):
  dinv = dinv_ref[...][:, :1]
  s = acc_ref[0] + acc_ref[1] + zp_ref[...]
  y = jnp.maximum(s * dinv + b_ref[...], 0.0)
  z = jnp.dot(y, w_ref[...], preferred_element_type=jnp.float32)
  out_ref[...] = z * dinv


def _tc_mid(acc, zp, dinv, W, b, f_out):
  return pl.pallas_call(
      _tc_mid_body,
      grid=(N // _B,),
      in_specs=[
          pl.BlockSpec((NC, _B, HID), lambda i: (0, i, 0)),
          pl.BlockSpec((_B, HID), lambda i: (i, 0)),
          pl.BlockSpec((_B, 16), lambda i: (i, 0)),
          pl.BlockSpec((HID, f_out), lambda i: (0, 0)),
          pl.BlockSpec((1, HID), lambda i: (0, 0)),
      ],
      out_specs=pl.BlockSpec((_B, f_out), lambda i: (i, 0)),
      out_shape=jax.ShapeDtypeStruct((N, f_out), jnp.float32),
  )(acc, zp, dinv, W, b)


def _tc_fin_body(acc_ref, zp_ref, dinv_ref, b_ref, out_ref):
  dinv = dinv_ref[...][:, :1]
  s = acc_ref[0] + acc_ref[1] + zp_ref[...]
  out_ref[...] = jnp.maximum(s * dinv + b_ref[...], 0.0)


def _tc_fin(acc, zp, dinv, b):
  return pl.pallas_call(
      _tc_fin_body,
      grid=(N // _B,),
      in_specs=[
          pl.BlockSpec((NC, _B, CLS), lambda i: (0, i, 0)),
          pl.BlockSpec((_B, CLS), lambda i: (i, 0)),
          pl.BlockSpec((_B, 16), lambda i: (i, 0)),
          pl.BlockSpec((1, CLS), lambda i: (0, 0)),
      ],
      out_specs=pl.BlockSpec((_B, CLS), lambda i: (i, 0)),
      out_shape=jax.ShapeDtypeStruct((N, CLS), jnp.float32),
  )(acc, zp, dinv, b)


def kernel(x, edge_index, W1, b1, W2, b2, W3, b3):
  pad_n = EPW_PAD - EPW
  src_pad = jnp.zeros((NW, pad_n), jnp.int32)
  dst_pad = (N + (jnp.arange(NW, dtype=jnp.int32) % NS)[:, None] * TRASH_PER_TILE
             + jnp.arange(pad_n, dtype=jnp.int32)[None, :] % TRASH_PER_TILE)
  src = jnp.concatenate(
      [edge_index[0].reshape(NW, EPW), src_pad], 1).reshape(NW, NCHUNK, K)
  dst = jnp.concatenate(
      [edge_index[1].reshape(NW, EPW), dst_pad], 1).reshape(NW, NCHUNK, K)
  zeros128 = jnp.zeros((N, HID), jnp.float32)
  zeros16 = jnp.zeros((N, 16), jnp.float32)

  degs = _sc_deg(dst, zeros16)
  z1p, dinv = _tc1(x, W1, degs)
  acc1 = _sc_agg128(z1p, src, dst, zeros128)
  z2p = _tc_mid(acc1, z1p, dinv, W2, b1.reshape(1, HID), HID)
  acc2 = _sc_agg128(z2p, src, dst, zeros128)
  z3p = _tc_mid(acc2, z2p, dinv, W3, b2.reshape(1, HID), CLS)
  acc3 = _sc_agg16(z3p, src, dst, zeros16)
  return _tc_fin(acc3, z3p, dinv, b3.reshape(1, CLS))


# speedup vs baseline: 2.3263x; 2.0596x over previous
import functools

import jax
import jax.numpy as jnp
from jax import lax
from jax.experimental import pallas as pl
from jax.experimental.pallas import tpu as pltpu
from jax.experimental.pallas import tpu_sc as plsc

N = 10000
E = 320000
D_IN = 128
HID = 128
CLS = 16

NC = 2
NS = 16
NW = NC * NS
EPW = E // NW
K = 128
NCHUNK = 80
EPW_PAD = NCHUNK * K
TRASH_PER_TILE = 80
NPAD = NS * TRASH_PER_TILE

RPT = 640
RPT_LAST = N - (NS - 1) * RPT


def _rows_copy(src_of, dst_of, sid):
  start = pl.multiple_of(sid * RPT, 8)

  @pl.when(sid < NS - 1)
  def _():
    pltpu.sync_copy(src_of(start, RPT), dst_of(start, RPT))

  @pl.when(sid == NS - 1)
  def _():
    pltpu.sync_copy(src_of(start, RPT_LAST), dst_of(start, RPT_LAST))

_MESH = plsc.VectorSubcoreMesh(
    core_axis_name="c", subcore_axis_name="s", num_cores=NC, num_subcores=NS
)
_SC_PARAMS = pltpu.CompilerParams(use_tc_tiling_on_sc=False)


def _make_sc_agg(F):
  @functools.partial(
      pl.kernel,
      out_type=jax.ShapeDtypeStruct((NC, N, F), jnp.float32),
      mesh=_MESH,
      compiler_params=_SC_PARAMS,
      scratch_types=[
          pltpu.VMEM((NCHUNK, K), jnp.int32),
          pltpu.VMEM((NCHUNK, K), jnp.int32),
          pltpu.VMEM((K, F), jnp.float32),
          pltpu.VMEM_SHARED((N + NPAD, F), jnp.float32),
          pltpu.SemaphoreType.DMA,
      ],
  )
  def agg(h_hbm, src_hbm, dst_hbm, zero_hbm, out_hbm,
          src_v, dst_v, rows_v, acc_sh, sem):
    cid = lax.axis_index("c")
    sid = lax.axis_index("s")
    wid = cid * NS + sid
    pltpu.sync_copy(src_hbm.at[wid], src_v)
    pltpu.sync_copy(dst_hbm.at[wid], dst_v)
    _rows_copy(lambda s, n: zero_hbm.at[pl.ds(s, n)],
               lambda s, n: acc_sh.at[pl.ds(s, n)], sid)
    plsc.subcore_barrier()

    def body(j, carry):
      pltpu.async_copy(h_hbm.at[src_v.at[j]], rows_v, sem).wait()
      pltpu.sync_copy(rows_v, acc_sh.at[dst_v.at[j]], add=True)
      return carry

    lax.fori_loop(0, NCHUNK, body, 0)
    plsc.subcore_barrier()
    _rows_copy(lambda s, n: acc_sh.at[pl.ds(s, n)],
               lambda s, n: out_hbm.at[cid].at[pl.ds(s, n)], sid)

  return agg


_sc_agg128 = _make_sc_agg(HID)
_sc_agg16 = _make_sc_agg(CLS)


@functools.partial(
    pl.kernel,
    out_type=jax.ShapeDtypeStruct((NC, N, 16), jnp.float32),
    mesh=_MESH,
    compiler_params=_SC_PARAMS,
    scratch_types=[
        pltpu.VMEM((NCHUNK, K), jnp.int32),
        pltpu.VMEM((K, 16), jnp.float32),
        pltpu.VMEM_SHARED((N + NPAD, 16), jnp.float32),
    ],
)
def _sc_deg(dst_hbm, zero_hbm, out_hbm, dst_v, ones_v, acc_sh):
  cid = lax.axis_index("c")
  sid = lax.axis_index("s")
  wid = cid * NS + sid
  pltpu.sync_copy(dst_hbm.at[wid], dst_v)
  _rows_copy(lambda s, n: zero_hbm.at[pl.ds(s, n)],
             lambda s, n: acc_sh.at[pl.ds(s, n)], sid)
  for j in range(K):
    ones_v[j, :] = jnp.full((16,), 1.0, jnp.float32)
  plsc.subcore_barrier()

  def body(j, carry):
    pltpu.sync_copy(ones_v, acc_sh.at[dst_v.at[j]], add=True)
    return carry

  lax.fori_loop(0, NCHUNK, body, 0)
  plsc.subcore_barrier()
  _rows_copy(lambda s, n: acc_sh.at[pl.ds(s, n)],
             lambda s, n: out_hbm.at[cid].at[pl.ds(s, n)], sid)


_B = 1000


def _tc1_body(x_ref, w_ref, deg_ref, z_ref, dinv_ref):
  deg = deg_ref[0] + deg_ref[1] + 1.0
  dinv = lax.rsqrt(deg)
  dinv_ref[...] = dinv
  z = jnp.dot(x_ref[...], w_ref[...], preferred_element_type=jnp.float32)
  z_ref[...] = z * dinv[:, :1]


def _tc1(x, W1, degs):
  return pl.pallas_call(
      _tc1_body,
      grid=(N // _B,),
      in_specs=[
          pl.BlockSpec((_B, D_IN), lambda i: (i, 0)),
          pl.BlockSpec((D_IN, HID), lambda i: (0, 0)),
          pl.BlockSpec((NC, _B, 16), lambda i: (0, i, 0)),
      ],
      out_specs=[
          pl.BlockSpec((_B, HID), lambda i: (i, 0)),
          pl.BlockSpec((_B, 16), lambda i: (i, 0)),
      ],
      out_shape=[
          jax.ShapeDtypeStruct((N, HID), jnp.float32),
          jax.ShapeDtypeStruct((N, 16), jnp.float32),
      ],
  )(x, W1, degs)


def _tc_mid_body(acc_ref, zp_ref, dinv_ref, w_ref, b_ref, out_ref):
  dinv = dinv_ref[...][:, :1]
  s = acc_ref[0] + acc_ref[1] + zp_ref[...]
  y = jnp.maximum(s * dinv + b_ref[...], 0.0)
  z = jnp.dot(y, w_ref[...], preferred_element_type=jnp.float32)
  out_ref[...] = z * dinv


def _tc_mid(acc, zp, dinv, W, b, f_out):
  return pl.pallas_call(
      _tc_mid_body,
      grid=(N // _B,),
      in_specs=[
          pl.BlockSpec((NC, _B, HID), lambda i: (0, i, 0)),
          pl.BlockSpec((_B, HID), lambda i: (i, 0)),
          pl.BlockSpec((_B, 16), lambda i: (i, 0)),
          pl.BlockSpec((HID, f_out), lambda i: (0, 0)),
          pl.BlockSpec((1, HID), lambda i: (0, 0)),
      ],
      out_specs=pl.BlockSpec((_B, f_out), lambda i: (i, 0)),
      out_shape=jax.ShapeDtypeStruct((N, f_out), jnp.float32),
  )(acc, zp, dinv, W, b)


def _tc_fin_body(acc_ref, zp_ref, dinv_ref, b_ref, out_ref):
  dinv = dinv_ref[...][:, :1]
  s = acc_ref[0] + acc_ref[1] + zp_ref[...]
  out_ref[...] = jnp.maximum(s * dinv + b_ref[...], 0.0)


def _tc_fin(acc, zp, dinv, b):
  return pl.pallas_call(
      _tc_fin_body,
      grid=(N // _B,),
      in_specs=[
          pl.BlockSpec((NC, _B, CLS), lambda i: (0, i, 0)),
          pl.BlockSpec((_B, CLS), lambda i: (i, 0)),
          pl.BlockSpec((_B, 16), lambda i: (i, 0)),
          pl.BlockSpec((1, CLS), lambda i: (0, 0)),
      ],
      out_specs=pl.BlockSpec((_B, CLS), lambda i: (i, 0)),
      out_shape=jax.ShapeDtypeStruct((N, CLS), jnp.float32),
  )(acc, zp, dinv, b)


def kernel(x, edge_index, W1, b1, W2, b2, W3, b3):
  pad_n = EPW_PAD - EPW
  src_pad = jnp.broadcast_to(
      jnp.arange(pad_n, dtype=jnp.int32)[None, :] * 37 % N, (NW, pad_n))
  dst_pad = (N + (jnp.arange(NW, dtype=jnp.int32) % NS)[:, None] * TRASH_PER_TILE
             + jnp.arange(pad_n, dtype=jnp.int32)[None, :] % TRASH_PER_TILE)
  src = jnp.concatenate(
      [edge_index[0].reshape(NW, EPW), src_pad], 1).reshape(NW, NCHUNK, K)
  dst = jnp.concatenate(
      [edge_index[1].reshape(NW, EPW), dst_pad], 1).reshape(NW, NCHUNK, K)
  zeros128 = jnp.zeros((N, HID), jnp.float32)
  zeros16 = jnp.zeros((N, 16), jnp.float32)

  degs = _sc_deg(dst, zeros16)
  z1p, dinv = _tc1(x, W1, degs)
  acc1 = _sc_agg128(z1p, src, dst, zeros128)
  z2p = _tc_mid(acc1, z1p, dinv, W2, b1.reshape(1, HID), HID)
  acc2 = _sc_agg128(z2p, src, dst, zeros128)
  z3p = _tc_mid(acc2, z2p, dinv, W3, b2.reshape(1, HID), CLS)
  acc3 = _sc_agg16(z3p, src, dst, zeros16)
  return _tc_fin(acc3, z3p, dinv, b3.reshape(1, CLS))


# speedup vs baseline: 2.3325x; 1.0027x over previous
import functools

import jax
import jax.numpy as jnp
from jax import lax
from jax.experimental import pallas as pl
from jax.experimental.pallas import tpu as pltpu
from jax.experimental.pallas import tpu_sc as plsc

N = 10000
E = 320000
D_IN = 128
HID = 128
CLS = 16

NC = 2
NS = 16
NW = NC * NS
EPW = E // NW
K = 128
NCHUNK = 80
EPW_PAD = NCHUNK * K
TRASH_PER_TILE = 1
NPAD = NS * TRASH_PER_TILE

RPT = 640
RPT_LAST = N - (NS - 1) * RPT


def _rows_copy(src_of, dst_of, sid):
  start = pl.multiple_of(sid * RPT, 8)

  @pl.when(sid < NS - 1)
  def _():
    pltpu.sync_copy(src_of(start, RPT), dst_of(start, RPT))

  @pl.when(sid == NS - 1)
  def _():
    pltpu.sync_copy(src_of(start, RPT_LAST), dst_of(start, RPT_LAST))

_MESH = plsc.VectorSubcoreMesh(
    core_axis_name="c", subcore_axis_name="s", num_cores=NC, num_subcores=NS
)
_SC_PARAMS = pltpu.CompilerParams(use_tc_tiling_on_sc=False)


def _make_sc_agg(F):
  @functools.partial(
      pl.kernel,
      out_type=jax.ShapeDtypeStruct((NC, N, F), jnp.float32),
      mesh=_MESH,
      compiler_params=_SC_PARAMS,
      scratch_types=[
          pltpu.VMEM((NCHUNK, K), jnp.int32),
          pltpu.VMEM((NCHUNK, K), jnp.int32),
          pltpu.VMEM((K, F), jnp.float32),
          pltpu.VMEM_SHARED((N + NPAD, F), jnp.float32),
          pltpu.SemaphoreType.DMA,
      ],
  )
  def agg(h_hbm, src_hbm, dst_hbm, zero_hbm, out_hbm,
          src_v, dst_v, rows_v, acc_sh, sem):
    cid = lax.axis_index("c")
    sid = lax.axis_index("s")
    wid = cid * NS + sid
    pltpu.sync_copy(src_hbm.at[wid], src_v)
    pltpu.sync_copy(dst_hbm.at[wid], dst_v)
    _rows_copy(lambda s, n: zero_hbm.at[pl.ds(s, n)],
               lambda s, n: acc_sh.at[pl.ds(s, n)], sid)
    plsc.subcore_barrier()

    def body(j, carry):
      pltpu.async_copy(h_hbm.at[src_v.at[j]], rows_v, sem).wait()
      pltpu.sync_copy(rows_v, acc_sh.at[dst_v.at[j]], add=True)
      return carry

    lax.fori_loop(0, NCHUNK, body, 0)
    plsc.subcore_barrier()
    _rows_copy(lambda s, n: acc_sh.at[pl.ds(s, n)],
               lambda s, n: out_hbm.at[cid].at[pl.ds(s, n)], sid)

  return agg


_sc_agg128 = _make_sc_agg(HID)
_sc_agg16 = _make_sc_agg(CLS)


@functools.partial(
    pl.kernel,
    out_type=jax.ShapeDtypeStruct((NC, N, 16), jnp.float32),
    mesh=_MESH,
    compiler_params=_SC_PARAMS,
    scratch_types=[
        pltpu.VMEM((NCHUNK, K), jnp.int32),
        pltpu.VMEM((K, 16), jnp.float32),
        pltpu.VMEM_SHARED((N + NPAD, 16), jnp.float32),
    ],
)
def _sc_deg(dst_hbm, zero_hbm, out_hbm, dst_v, ones_v, acc_sh):
  cid = lax.axis_index("c")
  sid = lax.axis_index("s")
  wid = cid * NS + sid
  pltpu.sync_copy(dst_hbm.at[wid], dst_v)
  _rows_copy(lambda s, n: zero_hbm.at[pl.ds(s, n)],
             lambda s, n: acc_sh.at[pl.ds(s, n)], sid)
  for j in range(K):
    ones_v[j, :] = jnp.full((16,), 1.0, jnp.float32)
  plsc.subcore_barrier()

  def body(j, carry):
    pltpu.sync_copy(ones_v, acc_sh.at[dst_v.at[j]], add=True)
    return carry

  lax.fori_loop(0, NCHUNK, body, 0)
  plsc.subcore_barrier()
  _rows_copy(lambda s, n: acc_sh.at[pl.ds(s, n)],
             lambda s, n: out_hbm.at[cid].at[pl.ds(s, n)], sid)


_B = 1000


def _tc1_body(x_ref, w_ref, deg_ref, z_ref, dinv_ref):
  deg = deg_ref[0] + deg_ref[1] + 1.0
  dinv = lax.rsqrt(deg)
  dinv_ref[...] = dinv
  z = jnp.dot(x_ref[...], w_ref[...], preferred_element_type=jnp.float32)
  z_ref[...] = z * dinv[:, :1]


def _tc1(x, W1, degs):
  return pl.pallas_call(
      _tc1_body,
      grid=(N // _B,),
      in_specs=[
          pl.BlockSpec((_B, D_IN), lambda i: (i, 0)),
          pl.BlockSpec((D_IN, HID), lambda i: (0, 0)),
          pl.BlockSpec((NC, _B, 16), lambda i: (0, i, 0)),
      ],
      out_specs=[
          pl.BlockSpec((_B, HID), lambda i: (i, 0)),
          pl.BlockSpec((_B, 16), lambda i: (i, 0)),
      ],
      out_shape=[
          jax.ShapeDtypeStruct((N, HID), jnp.float32),
          jax.ShapeDtypeStruct((N, 16), jnp.float32),
      ],
  )(x, W1, degs)


def _tc_mid_body(acc_ref, zp_ref, dinv_ref, w_ref, b_ref, out_ref):
  dinv = dinv_ref[...][:, :1]
  s = acc_ref[0] + acc_ref[1] + zp_ref[...]
  y = jnp.maximum(s * dinv + b_ref[...], 0.0)
  z = jnp.dot(y, w_ref[...], preferred_element_type=jnp.float32)
  out_ref[...] = z * dinv


def _tc_mid(acc, zp, dinv, W, b, f_out):
  return pl.pallas_call(
      _tc_mid_body,
      grid=(N // _B,),
      in_specs=[
          pl.BlockSpec((NC, _B, HID), lambda i: (0, i, 0)),
          pl.BlockSpec((_B, HID), lambda i: (i, 0)),
          pl.BlockSpec((_B, 16), lambda i: (i, 0)),
          pl.BlockSpec((HID, f_out), lambda i: (0, 0)),
          pl.BlockSpec((1, HID), lambda i: (0, 0)),
      ],
      out_specs=pl.BlockSpec((_B, f_out), lambda i: (i, 0)),
      out_shape=jax.ShapeDtypeStruct((N, f_out), jnp.float32),
  )(acc, zp, dinv, W, b)


def _tc_fin_body(acc_ref, zp_ref, dinv_ref, b_ref, out_ref):
  dinv = dinv_ref[...][:, :1]
  s = acc_ref[0] + acc_ref[1] + zp_ref[...]
  out_ref[...] = jnp.maximum(s * dinv + b_ref[...], 0.0)


def _tc_fin(acc, zp, dinv, b):
  return pl.pallas_call(
      _tc_fin_body,
      grid=(N // _B,),
      in_specs=[
          pl.BlockSpec((NC, _B, CLS), lambda i: (0, i, 0)),
          pl.BlockSpec((_B, CLS), lambda i: (i, 0)),
          pl.BlockSpec((_B, 16), lambda i: (i, 0)),
          pl.BlockSpec((1, CLS), lambda i: (0, 0)),
      ],
      out_specs=pl.BlockSpec((_B, CLS), lambda i: (i, 0)),
      out_shape=jax.ShapeDtypeStruct((N, CLS), jnp.float32),
  )(acc, zp, dinv, b)


def kernel(x, edge_index, W1, b1, W2, b2, W3, b3):
  pad_n = EPW_PAD - EPW
  src_pad = jnp.broadcast_to(
      jnp.arange(pad_n, dtype=jnp.int32)[None, :] * 37 % N, (NW, pad_n))
  dst_pad = (N + (jnp.arange(NW, dtype=jnp.int32) % NS)[:, None] * TRASH_PER_TILE
             + jnp.arange(pad_n, dtype=jnp.int32)[None, :] % TRASH_PER_TILE)
  src = jnp.concatenate(
      [edge_index[0].reshape(NW, EPW), src_pad], 1).reshape(NW, NCHUNK, K)
  dst = jnp.concatenate(
      [edge_index[1].reshape(NW, EPW), dst_pad], 1).reshape(NW, NCHUNK, K)
  zeros128 = jnp.zeros((N, HID), jnp.float32)
  zeros16 = jnp.zeros((N, 16), jnp.float32)

  degs = _sc_deg(dst, zeros16)
  z1p, dinv = _tc1(x, W1, degs)
  acc1 = _sc_agg128(z1p, src, dst, zeros128)
  z2p = _tc_mid(acc1, z1p, dinv, W2, b1.reshape(1, HID), HID)
  acc2 = _sc_agg128(z2p, src, dst, zeros128)
  z3p = _tc_mid(acc2, z2p, dinv, W3, b2.reshape(1, HID), CLS)
  acc3 = _sc_agg16(z3p, src, dst, zeros16)
  return _tc_fin(acc3, z3p, dinv, b3.reshape(1, CLS))


# speedup vs baseline: 3.1940x; 1.3693x over previous
import functools

import jax
import jax.numpy as jnp
from jax import lax
from jax.experimental import pallas as pl
from jax.experimental.pallas import tpu as pltpu
from jax.experimental.pallas import tpu_sc as plsc

N = 10000
E = 320000
D_IN = 128
HID = 128
CLS = 16

NC = 2
NS = 16
NW = NC * NS
EPW = E // NW
K = 96
NCHUNK = 106
EPW_PAD = NCHUNK * K
TRASH_PER_TILE = 1
NPAD = NS * TRASH_PER_TILE

RPT = 640
RPT_LAST = N - (NS - 1) * RPT


def _rows_copy(src_of, dst_of, sid):
  start = pl.multiple_of(sid * RPT, 8)

  @pl.when(sid < NS - 1)
  def _():
    pltpu.sync_copy(src_of(start, RPT), dst_of(start, RPT))

  @pl.when(sid == NS - 1)
  def _():
    pltpu.sync_copy(src_of(start, RPT_LAST), dst_of(start, RPT_LAST))

_MESH = plsc.VectorSubcoreMesh(
    core_axis_name="c", subcore_axis_name="s", num_cores=NC, num_subcores=NS
)
_SC_PARAMS = pltpu.CompilerParams(use_tc_tiling_on_sc=False)


def _make_sc_agg(F):
  @functools.partial(
      pl.kernel,
      out_type=jax.ShapeDtypeStruct((NC, N, F), jnp.float32),
      mesh=_MESH,
      compiler_params=_SC_PARAMS,
      scratch_types=[
          pltpu.VMEM((NCHUNK, K), jnp.int32),
          pltpu.VMEM((NCHUNK, K), jnp.int32),
          pltpu.VMEM((2, K, F), jnp.float32),
          pltpu.VMEM_SHARED((N + NPAD, F), jnp.float32),
          pltpu.SemaphoreType.DMA,
          pltpu.SemaphoreType.DMA,
      ],
  )
  def agg(h_hbm, src_hbm, dst_hbm, zero_hbm, out_hbm,
          src_v, dst_v, rows_v, acc_sh, sem0, sem1):
    cid = lax.axis_index("c")
    sid = lax.axis_index("s")
    wid = cid * NS + sid
    pltpu.sync_copy(src_hbm.at[wid], src_v)
    pltpu.sync_copy(dst_hbm.at[wid], dst_v)
    _rows_copy(lambda s, n: zero_hbm.at[pl.ds(s, n)],
               lambda s, n: acc_sh.at[pl.ds(s, n)], sid)
    plsc.subcore_barrier()

    rows0, rows1 = rows_v.at[0], rows_v.at[1]
    dummy = h_hbm.at[pl.ds(0, K)]
    pltpu.async_copy(h_hbm.at[src_v.at[0]], rows0, sem0)
    pltpu.async_copy(h_hbm.at[src_v.at[1]], rows1, sem1)

    def body(i, carry):
      j0 = 2 * i
      pltpu.make_async_copy(dummy, rows0, sem0).wait()
      pltpu.sync_copy(rows0, acc_sh.at[dst_v.at[j0]], add=True)

      @pl.when(j0 + 2 < NCHUNK)
      def _():
        pltpu.async_copy(h_hbm.at[src_v.at[j0 + 2]], rows0, sem0)

      pltpu.make_async_copy(dummy, rows1, sem1).wait()
      pltpu.sync_copy(rows1, acc_sh.at[dst_v.at[j0 + 1]], add=True)

      @pl.when(j0 + 3 < NCHUNK)
      def _():
        pltpu.async_copy(h_hbm.at[src_v.at[j0 + 3]], rows1, sem1)

      return carry

    lax.fori_loop(0, NCHUNK // 2, body, 0)
    plsc.subcore_barrier()
    _rows_copy(lambda s, n: acc_sh.at[pl.ds(s, n)],
               lambda s, n: out_hbm.at[cid].at[pl.ds(s, n)], sid)

  return agg


_sc_agg128 = _make_sc_agg(HID)
_sc_agg16 = _make_sc_agg(CLS)


@functools.partial(
    pl.kernel,
    out_type=jax.ShapeDtypeStruct((NC, N, 16), jnp.float32),
    mesh=_MESH,
    compiler_params=_SC_PARAMS,
    scratch_types=[
        pltpu.VMEM((NCHUNK, K), jnp.int32),
        pltpu.VMEM((K, 16), jnp.float32),
        pltpu.VMEM_SHARED((N + NPAD, 16), jnp.float32),
    ],
)
def _sc_deg(dst_hbm, zero_hbm, out_hbm, dst_v, ones_v, acc_sh):
  cid = lax.axis_index("c")
  sid = lax.axis_index("s")
  wid = cid * NS + sid
  pltpu.sync_copy(dst_hbm.at[wid], dst_v)
  _rows_copy(lambda s, n: zero_hbm.at[pl.ds(s, n)],
             lambda s, n: acc_sh.at[pl.ds(s, n)], sid)
  for j in range(K):
    ones_v[j, :] = jnp.full((16,), 1.0, jnp.float32)
  plsc.subcore_barrier()

  def body(j, carry):
    pltpu.sync_copy(ones_v, acc_sh.at[dst_v.at[j]], add=True)
    return carry

  lax.fori_loop(0, NCHUNK, body, 0)
  plsc.subcore_barrier()
  _rows_copy(lambda s, n: acc_sh.at[pl.ds(s, n)],
             lambda s, n: out_hbm.at[cid].at[pl.ds(s, n)], sid)


_B = 1000


def _tc1_body(x_ref, w_ref, deg_ref, z_ref, dinv_ref):
  deg = deg_ref[0] + deg_ref[1] + 1.0
  dinv = lax.rsqrt(deg)
  dinv_ref[...] = dinv
  z = jnp.dot(x_ref[...], w_ref[...], preferred_element_type=jnp.float32)
  z_ref[...] = z * dinv[:, :1]


def _tc1(x, W1, degs):
  return pl.pallas_call(
      _tc1_body,
      grid=(N // _B,),
      in_specs=[
          pl.BlockSpec((_B, D_IN), lambda i: (i, 0)),
          pl.BlockSpec((D_IN, HID), lambda i: (0, 0)),
          pl.BlockSpec((NC, _B, 16), lambda i: (0, i, 0)),
      ],
      out_specs=[
          pl.BlockSpec((_B, HID), lambda i: (i, 0)),
          pl.BlockSpec((_B, 16), lambda i: (i, 0)),
      ],
      out_shape=[
          jax.ShapeDtypeStruct((N, HID), jnp.float32),
          jax.ShapeDtypeStruct((N, 16), jnp.float32),
      ],
  )(x, W1, degs)


def _tc_mid_body(acc_ref, zp_ref, dinv_ref, w_ref, b_ref, out_ref):
  dinv = dinv_ref[...][:, :1]
  s = acc_ref[0] + acc_ref[1] + zp_ref[...]
  y = jnp.maximum(s * dinv + b_ref[...], 0.0)
  z = jnp.dot(y, w_ref[...], preferred_element_type=jnp.float32)
  out_ref[...] = z * dinv


def _tc_mid(acc, zp, dinv, W, b, f_out):
  return pl.pallas_call(
      _tc_mid_body,
      grid=(N // _B,),
      in_specs=[
          pl.BlockSpec((NC, _B, HID), lambda i: (0, i, 0)),
          pl.BlockSpec((_B, HID), lambda i: (i, 0)),
          pl.BlockSpec((_B, 16), lambda i: (i, 0)),
          pl.BlockSpec((HID, f_out), lambda i: (0, 0)),
          pl.BlockSpec((1, HID), lambda i: (0, 0)),
      ],
      out_specs=pl.BlockSpec((_B, f_out), lambda i: (i, 0)),
      out_shape=jax.ShapeDtypeStruct((N, f_out), jnp.float32),
  )(acc, zp, dinv, W, b)


def _tc_fin_body(acc_ref, zp_ref, dinv_ref, b_ref, out_ref):
  dinv = dinv_ref[...][:, :1]
  s = acc_ref[0] + acc_ref[1] + zp_ref[...]
  out_ref[...] = jnp.maximum(s * dinv + b_ref[...], 0.0)


def _tc_fin(acc, zp, dinv, b):
  return pl.pallas_call(
      _tc_fin_body,
      grid=(N // _B,),
      in_specs=[
          pl.BlockSpec((NC, _B, CLS), lambda i: (0, i, 0)),
          pl.BlockSpec((_B, CLS), lambda i: (i, 0)),
          pl.BlockSpec((_B, 16), lambda i: (i, 0)),
          pl.BlockSpec((1, CLS), lambda i: (0, 0)),
      ],
      out_specs=pl.BlockSpec((_B, CLS), lambda i: (i, 0)),
      out_shape=jax.ShapeDtypeStruct((N, CLS), jnp.float32),
  )(acc, zp, dinv, b)


def kernel(x, edge_index, W1, b1, W2, b2, W3, b3):
  pad_n = EPW_PAD - EPW
  src_pad = jnp.broadcast_to(
      jnp.arange(pad_n, dtype=jnp.int32)[None, :] * 37 % N, (NW, pad_n))
  dst_pad = (N + (jnp.arange(NW, dtype=jnp.int32) % NS)[:, None] * TRASH_PER_TILE
             + jnp.arange(pad_n, dtype=jnp.int32)[None, :] % TRASH_PER_TILE)
  src = jnp.concatenate(
      [edge_index[0].reshape(NW, EPW), src_pad], 1).reshape(NW, NCHUNK, K)
  dst = jnp.concatenate(
      [edge_index[1].reshape(NW, EPW), dst_pad], 1).reshape(NW, NCHUNK, K)
  zeros128 = jnp.zeros((N, HID), jnp.float32)
  zeros16 = jnp.zeros((N, 16), jnp.float32)

  degs = _sc_deg(dst, zeros16)
  z1p, dinv = _tc1(x, W1, degs)
  acc1 = _sc_agg128(z1p, src, dst, zeros128)
  z2p = _tc_mid(acc1, z1p, dinv, W2, b1.reshape(1, HID), HID)
  acc2 = _sc_agg128(z2p, src, dst, zeros128)
  z3p = _tc_mid(acc2, z2p, dinv, W3, b2.reshape(1, HID), CLS)
  acc3 = _sc_agg16(z3p, src, dst, zeros16)
  return _tc_fin(acc3, z3p, dinv, b3.reshape(1, CLS))


# speedup vs baseline: 3.3977x; 1.0638x over previous
import functools

import jax
import jax.numpy as jnp
from jax import lax
from jax.experimental import pallas as pl
from jax.experimental.pallas import tpu as pltpu
from jax.experimental.pallas import tpu_sc as plsc

N = 10000
E = 320000
D_IN = 128
HID = 128
CLS = 16

NC = 2
NS = 16
NW = NC * NS
EPW = E // NW
K = 64
NBUF = 3
NCHUNK = 159
EPW_PAD = NCHUNK * K
TRASH_PER_TILE = 1
NPAD = NS * TRASH_PER_TILE

RPT = 640
RPT_LAST = N - (NS - 1) * RPT


def _rows_copy(src_of, dst_of, sid):
  start = pl.multiple_of(sid * RPT, 8)

  @pl.when(sid < NS - 1)
  def _():
    pltpu.sync_copy(src_of(start, RPT), dst_of(start, RPT))

  @pl.when(sid == NS - 1)
  def _():
    pltpu.sync_copy(src_of(start, RPT_LAST), dst_of(start, RPT_LAST))

_MESH = plsc.VectorSubcoreMesh(
    core_axis_name="c", subcore_axis_name="s", num_cores=NC, num_subcores=NS
)
_SC_PARAMS = pltpu.CompilerParams(use_tc_tiling_on_sc=False)


def _make_sc_agg(F):
  @functools.partial(
      pl.kernel,
      out_type=jax.ShapeDtypeStruct((NC, N, F), jnp.float32),
      mesh=_MESH,
      compiler_params=_SC_PARAMS,
      scratch_types=[
          pltpu.VMEM((NCHUNK, K), jnp.int32),
          pltpu.VMEM((NCHUNK, K), jnp.int32),
          pltpu.VMEM((NBUF, K, F), jnp.float32),
          pltpu.VMEM_SHARED((N + NPAD, F), jnp.float32),
      ] + [pltpu.SemaphoreType.DMA] * NBUF,
  )
  def agg(h_hbm, src_hbm, dst_hbm, zero_hbm, out_hbm,
          src_v, dst_v, rows_v, acc_sh, *sems):
    cid = lax.axis_index("c")
    sid = lax.axis_index("s")
    wid = cid * NS + sid
    pltpu.sync_copy(src_hbm.at[wid], src_v)
    pltpu.sync_copy(dst_hbm.at[wid], dst_v)
    _rows_copy(lambda s, n: zero_hbm.at[pl.ds(s, n)],
               lambda s, n: acc_sh.at[pl.ds(s, n)], sid)
    plsc.subcore_barrier()

    dummy = h_hbm.at[pl.ds(0, K)]
    for b in range(NBUF):
      pltpu.async_copy(h_hbm.at[src_v.at[b]], rows_v.at[b], sems[b])

    def body(i, carry):
      j0 = NBUF * i
      for b in range(NBUF):
        j = j0 + b
        pltpu.make_async_copy(dummy, rows_v.at[b], sems[b]).wait()
        pltpu.sync_copy(rows_v.at[b], acc_sh.at[dst_v.at[j]], add=True)

        @pl.when(j + NBUF < NCHUNK)
        def _(b=b, j=j):
          pltpu.async_copy(h_hbm.at[src_v.at[j + NBUF]], rows_v.at[b], sems[b])

      return carry

    lax.fori_loop(0, NCHUNK // NBUF, body, 0)
    plsc.subcore_barrier()
    _rows_copy(lambda s, n: acc_sh.at[pl.ds(s, n)],
               lambda s, n: out_hbm.at[cid].at[pl.ds(s, n)], sid)

  return agg


_sc_agg128 = _make_sc_agg(HID)
_sc_agg16 = _make_sc_agg(CLS)


@functools.partial(
    pl.kernel,
    out_type=jax.ShapeDtypeStruct((NC, N, 16), jnp.float32),
    mesh=_MESH,
    compiler_params=_SC_PARAMS,
    scratch_types=[
        pltpu.VMEM((NCHUNK, K), jnp.int32),
        pltpu.VMEM((K, 16), jnp.float32),
        pltpu.VMEM_SHARED((N + NPAD, 16), jnp.float32),
    ],
)
def _sc_deg(dst_hbm, zero_hbm, out_hbm, dst_v, ones_v, acc_sh):
  cid = lax.axis_index("c")
  sid = lax.axis_index("s")
  wid = cid * NS + sid
  pltpu.sync_copy(dst_hbm.at[wid], dst_v)
  _rows_copy(lambda s, n: zero_hbm.at[pl.ds(s, n)],
             lambda s, n: acc_sh.at[pl.ds(s, n)], sid)
  for j in range(K):
    ones_v[j, :] = jnp.full((16,), 1.0, jnp.float32)
  plsc.subcore_barrier()

  def body(j, carry):
    pltpu.sync_copy(ones_v, acc_sh.at[dst_v.at[j]], add=True)
    return carry

  lax.fori_loop(0, NCHUNK, body, 0)
  plsc.subcore_barrier()
  _rows_copy(lambda s, n: acc_sh.at[pl.ds(s, n)],
             lambda s, n: out_hbm.at[cid].at[pl.ds(s, n)], sid)


_B = 1000


def _tc1_body(x_ref, w_ref, deg_ref, z_ref, dinv_ref):
  deg = deg_ref[0] + deg_ref[1] + 1.0
  dinv = lax.rsqrt(deg)
  dinv_ref[...] = dinv
  z = jnp.dot(x_ref[...], w_ref[...], preferred_element_type=jnp.float32)
  z_ref[...] = z * dinv[:, :1]


def _tc1(x, W1, degs):
  return pl.pallas_call(
      _tc1_body,
      grid=(N // _B,),
      in_specs=[
          pl.BlockSpec((_B, D_IN), lambda i: (i, 0)),
          pl.BlockSpec((D_IN, HID), lambda i: (0, 0)),
          pl.BlockSpec((NC, _B, 16), lambda i: (0, i, 0)),
      ],
      out_specs=[
          pl.BlockSpec((_B, HID), lambda i: (i, 0)),
          pl.BlockSpec((_B, 16), lambda i: (i, 0)),
      ],
      out_shape=[
          jax.ShapeDtypeStruct((N, HID), jnp.float32),
          jax.ShapeDtypeStruct((N, 16), jnp.float32),
      ],
  )(x, W1, degs)


def _tc_mid_body(acc_ref, zp_ref, dinv_ref, w_ref, b_ref, out_ref):
  dinv = dinv_ref[...][:, :1]
  s = acc_ref[0] + acc_ref[1] + zp_ref[...]
  y = jnp.maximum(s * dinv + b_ref[...], 0.0)
  z = jnp.dot(y, w_ref[...], preferred_element_type=jnp.float32)
  out_ref[...] = z * dinv


def _tc_mid(acc, zp, dinv, W, b, f_out):
  return pl.pallas_call(
      _tc_mid_body,
      grid=(N // _B,),
      in_specs=[
          pl.BlockSpec((NC, _B, HID), lambda i: (0, i, 0)),
          pl.BlockSpec((_B, HID), lambda i: (i, 0)),
          pl.BlockSpec((_B, 16), lambda i: (i, 0)),
          pl.BlockSpec((HID, f_out), lambda i: (0, 0)),
          pl.BlockSpec((1, HID), lambda i: (0, 0)),
      ],
      out_specs=pl.BlockSpec((_B, f_out), lambda i: (i, 0)),
      out_shape=jax.ShapeDtypeStruct((N, f_out), jnp.float32),
  )(acc, zp, dinv, W, b)


def _tc_fin_body(acc_ref, zp_ref, dinv_ref, b_ref, out_ref):
  dinv = dinv_ref[...][:, :1]
  s = acc_ref[0] + acc_ref[1] + zp_ref[...]
  out_ref[...] = jnp.maximum(s * dinv + b_ref[...], 0.0)


def _tc_fin(acc, zp, dinv, b):
  return pl.pallas_call(
      _tc_fin_body,
      grid=(N // _B,),
      in_specs=[
          pl.BlockSpec((NC, _B, CLS), lambda i: (0, i, 0)),
          pl.BlockSpec((_B, CLS), lambda i: (i, 0)),
          pl.BlockSpec((_B, 16), lambda i: (i, 0)),
          pl.BlockSpec((1, CLS), lambda i: (0, 0)),
      ],
      out_specs=pl.BlockSpec((_B, CLS), lambda i: (i, 0)),
      out_shape=jax.ShapeDtypeStruct((N, CLS), jnp.float32),
  )(acc, zp, dinv, b)


def kernel(x, edge_index, W1, b1, W2, b2, W3, b3):
  pad_n = EPW_PAD - EPW
  src_pad = jnp.broadcast_to(
      jnp.arange(pad_n, dtype=jnp.int32)[None, :] * 37 % N, (NW, pad_n))
  dst_pad = (N + (jnp.arange(NW, dtype=jnp.int32) % NS)[:, None] * TRASH_PER_TILE
             + jnp.arange(pad_n, dtype=jnp.int32)[None, :] % TRASH_PER_TILE)
  src = jnp.concatenate(
      [edge_index[0].reshape(NW, EPW), src_pad], 1).reshape(NW, NCHUNK, K)
  dst = jnp.concatenate(
      [edge_index[1].reshape(NW, EPW), dst_pad], 1).reshape(NW, NCHUNK, K)
  zeros128 = jnp.zeros((N, HID), jnp.float32)
  zeros16 = jnp.zeros((N, 16), jnp.float32)

  degs = _sc_deg(dst, zeros16)
  z1p, dinv = _tc1(x, W1, degs)
  acc1 = _sc_agg128(z1p, src, dst, zeros128)
  z2p = _tc_mid(acc1, z1p, dinv, W2, b1.reshape(1, HID), HID)
  acc2 = _sc_agg128(z2p, src, dst, zeros128)
  z3p = _tc_mid(acc2, z2p, dinv, W3, b2.reshape(1, HID), CLS)
  acc3 = _sc_agg16(z3p, src, dst, zeros16)
  return _tc_fin(acc3, z3p, dinv, b3.reshape(1, CLS))
